# Initial kernel scaffold; baseline (speedup 1.0000x reference)
import jax
import jax.numpy as jnp
from jax.experimental import pallas as pl


def kernel(x, adj, W, b):
    raise NotImplementedError("write your pallas kernel here")


# speedup vs baseline: 1.3316x; 1.3316x over previous
import functools

import jax
import jax.numpy as jnp
from jax import lax
from jax.experimental import pallas as pl
from jax.experimental.pallas import tpu as pltpu
from jax.experimental.pallas import tpu_sc as plsc

N = 10000
DEG = 32
IN_DIM = 128
HIDDEN = 128

NUM_CORES = 2
NUM_SUBCORES = 16
NW = NUM_CORES * NUM_SUBCORES
NODES_PER_W = 320
NPAD = NW * NODES_PER_W
CHUNK = 4
IDX_PER_CHUNK = CHUNK * DEG
NCH = NODES_PER_W // CHUNK
LANES = 16
GROUPS = IN_DIM // LANES


def _tree_sum(vals):
    while len(vals) > 1:
        nxt = [vals[i] + vals[i + 1] for i in range(0, len(vals) - 1, 2)]
        if len(vals) % 2:
            nxt.append(vals[-1])
        vals = nxt
    return vals[0]


def _sc_mean_agg(x, idx_flat):
    mesh = plsc.VectorSubcoreMesh(core_axis_name="c", subcore_axis_name="s")

    @functools.partial(
        pl.kernel,
        out_type=jax.ShapeDtypeStruct((NPAD, IN_DIM), jnp.float32),
        mesh=mesh,
        scratch_types=[
            pltpu.VMEM((IDX_PER_CHUNK,), jnp.int32),
            pltpu.VMEM((IDX_PER_CHUNK,), jnp.int32),
            pltpu.VMEM((IDX_PER_CHUNK, IN_DIM), jnp.float32),
            pltpu.VMEM((IDX_PER_CHUNK, IN_DIM), jnp.float32),
            pltpu.VMEM((NODES_PER_W, IN_DIM), jnp.float32),
            pltpu.SemaphoreType.DMA,
            pltpu.SemaphoreType.DMA,
        ],
    )
    def k(x_hbm, idx_hbm, out_hbm, idx0, idx1, rows0, rows1, outbuf, g0, g1):
        wid = lax.axis_index("c") * NUM_SUBCORES + lax.axis_index("s")
        ibase = wid * NODES_PER_W * DEG

        def fire(c, idxv, rowsv, gsem):
            pltpu.sync_copy(idx_hbm.at[pl.ds(ibase + c * IDX_PER_CHUNK,
                                             IDX_PER_CHUNK)], idxv)
            pltpu.async_copy(x_hbm.at[idxv], rowsv, gsem)

        def wait_gather(idxv, rowsv, gsem):
            pltpu.make_async_copy(x_hbm.at[idxv], rowsv, gsem).wait()

        def reduce_chunk(c, rowsv):
            inv = jnp.float32(1.0 / DEG)
            for n in range(CHUNK):
                for g in range(GROUPS):
                    sl = pl.ds(g * LANES, LANES)
                    vals = [rowsv[n * DEG + j, sl] for j in range(DEG)]
                    outbuf[c * CHUNK + n, sl] = _tree_sum(vals) * inv

        fire(0, idx0, rows0, g0)

        @pl.loop(0, NCH, step=2)
        def _(c):
            fire(c + 1, idx1, rows1, g1)
            wait_gather(idx0, rows0, g0)
            reduce_chunk(c, rows0)

            @pl.when(c + 2 < NCH)
            def _():
                fire(c + 2, idx0, rows0, g0)

            wait_gather(idx1, rows1, g1)
            reduce_chunk(c + 1, rows1)

        pltpu.sync_copy(outbuf, out_hbm.at[pl.ds(wid * NODES_PER_W,
                                                 NODES_PER_W)])

    return k(x, idx_flat)


def _tc_linear(x, agg, W, b2):
    BLK = 2000

    def body(x_ref, a_ref, w_ref, b_ref, o_ref):
        w = w_ref[...]
        h1 = lax.dot_general(x_ref[...], w[:, :IN_DIM],
                             (((1,), (1,)), ((), ())),
                             precision=lax.Precision.HIGHEST,
                             preferred_element_type=jnp.float32)
        h2 = lax.dot_general(a_ref[...], w[:, IN_DIM:],
                             (((1,), (1,)), ((), ())),
                             precision=lax.Precision.HIGHEST,
                             preferred_element_type=jnp.float32)
        o_ref[...] = jnp.maximum(h1 + h2 + b_ref[...], 0.0)

    return pl.pallas_call(
        body,
        grid=(N // BLK,),
        in_specs=[
            pl.BlockSpec((BLK, IN_DIM), lambda i: (i, 0)),
            pl.BlockSpec((BLK, IN_DIM), lambda i: (i, 0)),
            pl.BlockSpec((HIDDEN, 2 * IN_DIM), lambda i: (0, 0)),
            pl.BlockSpec((1, HIDDEN), lambda i: (0, 0)),
        ],
        out_specs=pl.BlockSpec((BLK, HIDDEN), lambda i: (i, 0)),
        out_shape=jax.ShapeDtypeStruct((N, HIDDEN), jnp.float32),
    )(x, agg, W, b2)


def kernel(x, adj, W, b):
    idx = adj.astype(jnp.int32).reshape(-1)
    idx = jnp.concatenate(
        [idx, jnp.zeros((NPAD - N) * DEG, dtype=jnp.int32)])
    agg = _sc_mean_agg(x, idx)[:N]
    return _tc_linear(x, agg, W, b.reshape(1, HIDDEN))


# speedup vs baseline: 1.3439x; 1.0092x over previous
import functools

import jax
import jax.numpy as jnp
from jax import lax
from jax.experimental import pallas as pl
from jax.experimental.pallas import tpu as pltpu
from jax.experimental.pallas import tpu_sc as plsc

N = 10000
DEG = 32
IN_DIM = 128
HIDDEN = 128

NUM_CORES = 2
NUM_SUBCORES = 16
NW = NUM_CORES * NUM_SUBCORES
NODES_PER_W = 320
NPAD = NW * NODES_PER_W
CHUNK = 4
IDX_PER_CHUNK = CHUNK * DEG
NCH = NODES_PER_W // CHUNK
LANES = 16
GROUPS = IN_DIM // LANES


def _tree_sum(vals):
    while len(vals) > 1:
        nxt = [vals[i] + vals[i + 1] for i in range(0, len(vals) - 1, 2)]
        if len(vals) % 2:
            nxt.append(vals[-1])
        vals = nxt
    return vals[0]


def _sc_mean_agg(x, idx_flat):
    mesh = plsc.VectorSubcoreMesh(core_axis_name="c", subcore_axis_name="s")

    NBUF = 4

    @functools.partial(
        pl.kernel,
        out_type=jax.ShapeDtypeStruct((NPAD, IN_DIM), jnp.float32),
        mesh=mesh,
        scratch_types=[
            pltpu.VMEM((NODES_PER_W * DEG,), jnp.int32),
            [pltpu.VMEM((IDX_PER_CHUNK, IN_DIM), jnp.float32)
             for _ in range(NBUF)],
            pltpu.VMEM((NODES_PER_W, IN_DIM), jnp.float32),
            [pltpu.SemaphoreType.DMA for _ in range(NBUF)],
        ],
    )
    def k(x_hbm, idx_hbm, out_hbm, idx_all, rows, outbuf, gsems):
        wid = lax.axis_index("c") * NUM_SUBCORES + lax.axis_index("s")

        pltpu.sync_copy(idx_hbm.at[pl.ds(wid * NODES_PER_W * DEG,
                                         NODES_PER_W * DEG)], idx_all)

        def fire(c, k_slot):
            idxv = idx_all.at[pl.ds(c * IDX_PER_CHUNK, IDX_PER_CHUNK)]
            pltpu.async_copy(x_hbm.at[idxv], rows[k_slot], gsems[k_slot])

        def wait_gather(c, k_slot):
            idxv = idx_all.at[pl.ds(c * IDX_PER_CHUNK, IDX_PER_CHUNK)]
            pltpu.make_async_copy(x_hbm.at[idxv], rows[k_slot],
                                  gsems[k_slot]).wait()

        def reduce_chunk(c, rowsv):
            inv = jnp.float32(1.0 / DEG)
            for n in range(CHUNK):
                for g in range(GROUPS):
                    sl = pl.ds(g * LANES, LANES)
                    vals = [rowsv[n * DEG + j, sl] for j in range(DEG)]
                    outbuf[c * CHUNK + n, sl] = _tree_sum(vals) * inv

        for k_slot in range(NBUF):
            fire(k_slot, k_slot)

        @pl.loop(0, NCH, step=NBUF)
        def _(c):
            for k_slot in range(NBUF):
                wait_gather(c + k_slot, k_slot)
                reduce_chunk(c + k_slot, rows[k_slot])

                @pl.when(c + k_slot + NBUF < NCH)
                def _():
                    fire(c + k_slot + NBUF, k_slot)

        pltpu.sync_copy(outbuf, out_hbm.at[pl.ds(wid * NODES_PER_W,
                                                 NODES_PER_W)])

    return k(x, idx_flat)


def _tc_linear(x, agg, W, b2):
    BLK = 2000

    def body(x_ref, a_ref, w_ref, b_ref, o_ref):
        w = w_ref[...]
        h1 = lax.dot_general(x_ref[...], w[:, :IN_DIM],
                             (((1,), (1,)), ((), ())),
                             precision=lax.Precision.HIGHEST,
                             preferred_element_type=jnp.float32)
        h2 = lax.dot_general(a_ref[...], w[:, IN_DIM:],
                             (((1,), (1,)), ((), ())),
                             precision=lax.Precision.HIGHEST,
                             preferred_element_type=jnp.float32)
        o_ref[...] = jnp.maximum(h1 + h2 + b_ref[...], 0.0)

    return pl.pallas_call(
        body,
        grid=(N // BLK,),
        in_specs=[
            pl.BlockSpec((BLK, IN_DIM), lambda i: (i, 0)),
            pl.BlockSpec((BLK, IN_DIM), lambda i: (i, 0)),
            pl.BlockSpec((HIDDEN, 2 * IN_DIM), lambda i: (0, 0)),
            pl.BlockSpec((1, HIDDEN), lambda i: (0, 0)),
        ],
        out_specs=pl.BlockSpec((BLK, HIDDEN), lambda i: (i, 0)),
        out_shape=jax.ShapeDtypeStruct((N, HIDDEN), jnp.float32),
    )(x, agg, W, b2)


def kernel(x, adj, W, b):
    idx = adj.astype(jnp.int32).reshape(-1)
    idx = jnp.concatenate(
        [idx, jnp.zeros((NPAD - N) * DEG, dtype=jnp.int32)])
    agg = _sc_mean_agg(x, idx)[:N]
    return _tc_linear(x, agg, W, b.reshape(1, HIDDEN))


# speedup vs baseline: 2.8889x; 2.1496x over previous
import functools

import jax
import jax.numpy as jnp
from jax import lax
from jax.experimental import pallas as pl
from jax.experimental.pallas import tpu as pltpu
from jax.experimental.pallas import tpu_sc as plsc

N = 10000
DEG = 32
IN_DIM = 128
HIDDEN = 128

NUM_CORES = 2
NUM_SUBCORES = 16
NW = NUM_CORES * NUM_SUBCORES
NODES_PER_W = 320
NPAD = NW * NODES_PER_W
CHUNK = 4
IDX_PER_CHUNK = CHUNK * DEG
NCH = NODES_PER_W // CHUNK
LANES = 16
GROUPS = IN_DIM // LANES


def _tree_sum(vals):
    while len(vals) > 1:
        nxt = [vals[i] + vals[i + 1] for i in range(0, len(vals) - 1, 2)]
        if len(vals) % 2:
            nxt.append(vals[-1])
        vals = nxt
    return vals[0]


def _sc_mean_agg(x, idx_flat):
    mesh = plsc.VectorSubcoreMesh(core_axis_name="c", subcore_axis_name="s")

    NBUF = 4

    @functools.partial(
        pl.kernel,
        out_type=jax.ShapeDtypeStruct((NPAD, IN_DIM), jnp.float32),
        mesh=mesh,
        scratch_types=[
            pltpu.VMEM((NODES_PER_W * DEG,), jnp.int32),
            [pltpu.VMEM((IDX_PER_CHUNK, IN_DIM), jnp.float32)
             for _ in range(NBUF)],
            pltpu.VMEM((NODES_PER_W, IN_DIM), jnp.float32),
            [pltpu.SemaphoreType.DMA for _ in range(NBUF)],
        ],
    )
    def k(x_hbm, idx_hbm, out_hbm, idx_all, rows, outbuf, gsems):
        wid = lax.axis_index("c") * NUM_SUBCORES + lax.axis_index("s")

        pltpu.sync_copy(idx_hbm.at[pl.ds(wid * NODES_PER_W * DEG,
                                         NODES_PER_W * DEG)], idx_all)

        def fire(c, k_slot):
            idxv = idx_all.at[pl.ds(c * IDX_PER_CHUNK, IDX_PER_CHUNK)]
            pltpu.async_copy(x_hbm.at[idxv], rows[k_slot], gsems[k_slot])

        def wait_gather(c, k_slot):
            idxv = idx_all.at[pl.ds(c * IDX_PER_CHUNK, IDX_PER_CHUNK)]
            pltpu.make_async_copy(x_hbm.at[idxv], rows[k_slot],
                                  gsems[k_slot]).wait()

        def reduce_chunk(c, rowsv):
            inv = jnp.float32(1.0 / DEG)
            for n in range(CHUNK):
                for g in range(GROUPS):
                    sl = pl.ds(g * LANES, LANES)
                    vals = [rowsv[n * DEG + j, sl] for j in range(DEG)]
                    outbuf[c * CHUNK + n, sl] = _tree_sum(vals) * inv

        for k_slot in range(NBUF):
            fire(k_slot, k_slot)

        @pl.loop(0, NCH, step=NBUF)
        def _(c):
            for k_slot in range(NBUF):
                wait_gather(c + k_slot, k_slot)
                reduce_chunk(c + k_slot, rows[k_slot])

                @pl.when(c + k_slot + NBUF < NCH)
                def _():
                    fire(c + k_slot + NBUF, k_slot)

        pltpu.sync_copy(outbuf, out_hbm.at[pl.ds(wid * NODES_PER_W,
                                                 NODES_PER_W)])

    return k(x, idx_flat)


def _tc_linear(x, agg, W, b2):
    BLK = 2000

    def body(x_ref, a_ref, w_ref, b_ref, o_ref):
        w = w_ref[...]
        h1 = lax.dot_general(x_ref[...], w[:, :IN_DIM],
                             (((1,), (1,)), ((), ())),
                             precision=lax.Precision.HIGHEST,
                             preferred_element_type=jnp.float32)
        h2 = lax.dot_general(a_ref[...], w[:, IN_DIM:],
                             (((1,), (1,)), ((), ())),
                             precision=lax.Precision.HIGHEST,
                             preferred_element_type=jnp.float32)
        o_ref[...] = jnp.maximum(h1 + h2 + b_ref[...], 0.0)

    return pl.pallas_call(
        body,
        grid=(N // BLK,),
        in_specs=[
            pl.BlockSpec((BLK, IN_DIM), lambda i: (i, 0)),
            pl.BlockSpec((BLK, IN_DIM), lambda i: (i, 0)),
            pl.BlockSpec((HIDDEN, 2 * IN_DIM), lambda i: (0, 0)),
            pl.BlockSpec((1, HIDDEN), lambda i: (0, 0)),
        ],
        out_specs=pl.BlockSpec((BLK, HIDDEN), lambda i: (i, 0)),
        out_shape=jax.ShapeDtypeStruct((N, HIDDEN), jnp.float32),
    )(x, agg, W, b2)


def kernel(x, adj, W, b):
    idx = adj.astype(jnp.int32).reshape(-1)
    pad = jnp.arange((NPAD - N) * DEG, dtype=jnp.int32) % N
    idx = jnp.concatenate([idx, pad])
    agg = _sc_mean_agg(x, idx)[:N]
    return _tc_linear(x, agg, W, b.reshape(1, HIDDEN))


# speedup vs baseline: 6.3747x; 2.2066x over previous
import functools

import jax
import jax.numpy as jnp
from jax import lax
from jax.experimental import pallas as pl
from jax.experimental.pallas import tpu as pltpu
from jax.experimental.pallas import tpu_sc as plsc

N = 10000
DEG = 32
IN_DIM = 128
HIDDEN = 128

NUM_CORES = 2
NUM_SUBCORES = 16
NW = NUM_CORES * NUM_SUBCORES
NODES_PER_W = 320
NPAD = NW * NODES_PER_W
CHUNK = 4
IDX_PER_CHUNK = CHUNK * DEG
NCH = NODES_PER_W // CHUNK
LANES = 16
GROUPS = IN_DIM // LANES


def _tree_sum(vals):
    while len(vals) > 1:
        nxt = [vals[i] + vals[i + 1] for i in range(0, len(vals) - 1, 2)]
        if len(vals) % 2:
            nxt.append(vals[-1])
        vals = nxt
    return vals[0]


def _sc_neighbor_sum(x, idx_flat):
    mesh = plsc.VectorSubcoreMesh(core_axis_name="c", subcore_axis_name="s")
    NBUF = 4

    @functools.partial(
        pl.kernel,
        out_type=jax.ShapeDtypeStruct((NPAD, IN_DIM), jnp.float32),
        mesh=mesh,
        scratch_types=[
            pltpu.VMEM((NODES_PER_W * DEG,), jnp.int32),
            [pltpu.VMEM((IDX_PER_CHUNK, IN_DIM), jnp.float32)
             for _ in range(NBUF)],
            pltpu.VMEM((NODES_PER_W, IN_DIM), jnp.float32),
            [pltpu.SemaphoreType.DMA for _ in range(NBUF)],
        ],
    )
    def k(x_hbm, idx_hbm, out_hbm, idx_all, rows, outbuf, gsems):
        wid = lax.axis_index("c") * NUM_SUBCORES + lax.axis_index("s")

        pltpu.sync_copy(idx_hbm.at[pl.ds(wid * NODES_PER_W * DEG,
                                         NODES_PER_W * DEG)], idx_all)

        def fire(c, k_slot):
            idxv = idx_all.at[pl.ds(c * IDX_PER_CHUNK, IDX_PER_CHUNK)]
            pltpu.async_copy(x_hbm.at[idxv], rows[k_slot], gsems[k_slot])

        def wait_gather(c, k_slot):
            idxv = idx_all.at[pl.ds(c * IDX_PER_CHUNK, IDX_PER_CHUNK)]
            pltpu.make_async_copy(x_hbm.at[idxv], rows[k_slot],
                                  gsems[k_slot]).wait()

        def reduce_chunk(c, rowsv):
            @pl.loop(0, CHUNK)
            def _(n):
                base = n * DEG
                for g in range(GROUPS):
                    sl = pl.ds(g * LANES, LANES)
                    vals = [rowsv[base + j, sl] for j in range(DEG)]
                    outbuf[c * CHUNK + n, sl] = _tree_sum(vals)

        for k_slot in range(NBUF):
            fire(k_slot, k_slot)

        @pl.loop(0, NCH, step=NBUF)
        def _(c):
            for k_slot in range(NBUF):
                wait_gather(c + k_slot, k_slot)
                reduce_chunk(c + k_slot, rows[k_slot])

                @pl.when(c + k_slot + NBUF < NCH)
                def _():
                    fire(c + k_slot + NBUF, k_slot)

        pltpu.sync_copy(outbuf, out_hbm.at[pl.ds(wid * NODES_PER_W,
                                                 NODES_PER_W)])

    return k(x, idx_flat)


def _tc_linear(x, aggsum, W, b2):
    BLK = 2000

    def body(x_ref, a_ref, w_ref, b_ref, o_ref):
        w = w_ref[...]
        h1 = lax.dot_general(x_ref[...], w[:, :IN_DIM],
                             (((1,), (1,)), ((), ())),
                             precision=lax.Precision.HIGHEST,
                             preferred_element_type=jnp.float32)
        h2 = lax.dot_general(a_ref[...], w[:, IN_DIM:],
                             (((1,), (1,)), ((), ())),
                             precision=lax.Precision.HIGHEST,
                             preferred_element_type=jnp.float32)
        o_ref[...] = jnp.maximum(h1 + h2 * (1.0 / DEG) + b_ref[...], 0.0)

    return pl.pallas_call(
        body,
        grid=(N // BLK,),
        in_specs=[
            pl.BlockSpec((BLK, IN_DIM), lambda i: (i, 0)),
            pl.BlockSpec((BLK, IN_DIM), lambda i: (i, 0)),
            pl.BlockSpec((HIDDEN, 2 * IN_DIM), lambda i: (0, 0)),
            pl.BlockSpec((1, HIDDEN), lambda i: (0, 0)),
        ],
        out_specs=pl.BlockSpec((BLK, HIDDEN), lambda i: (i, 0)),
        out_shape=jax.ShapeDtypeStruct((N, HIDDEN), jnp.float32),
    )(x, aggsum, W, b2)


def kernel(x, adj, W, b):
    idx = adj.astype(jnp.int32).reshape(-1)
    pad = jnp.arange((NPAD - N) * DEG, dtype=jnp.int32) % N
    idx = jnp.concatenate([idx, pad])
    aggsum = _sc_neighbor_sum(x, idx)[:N]
    return _tc_linear(x, aggsum, W, b.reshape(1, HIDDEN))


# speedup vs baseline: 6.6110x; 1.0371x over previous
import functools

import jax
import jax.numpy as jnp
from jax import lax
from jax.experimental import pallas as pl
from jax.experimental.pallas import tpu as pltpu
from jax.experimental.pallas import tpu_sc as plsc

N = 10000
DEG = 32
IN_DIM = 128
HIDDEN = 128

NUM_CORES = 2
NUM_SUBCORES = 16
NW = NUM_CORES * NUM_SUBCORES
NODES_PER_W = 320
NPAD = NW * NODES_PER_W
CHUNK = 4
IDX_PER_CHUNK = CHUNK * DEG
NCH = NODES_PER_W // CHUNK
LANES = 16
GROUPS = IN_DIM // LANES


def _tree_sum(vals):
    while len(vals) > 1:
        nxt = [vals[i] + vals[i + 1] for i in range(0, len(vals) - 1, 2)]
        if len(vals) % 2:
            nxt.append(vals[-1])
        vals = nxt
    return vals[0]


def _sc_neighbor_sum(x, idx_flat):
    mesh = plsc.VectorSubcoreMesh(core_axis_name="c", subcore_axis_name="s")
    NBUF = 4

    @functools.partial(
        pl.kernel,
        out_type=jax.ShapeDtypeStruct((NPAD, IN_DIM), jnp.float32),
        mesh=mesh,
        scratch_types=[
            pltpu.VMEM((NODES_PER_W * DEG,), jnp.int32),
            [pltpu.VMEM((IDX_PER_CHUNK, IN_DIM), jnp.float32)
             for _ in range(NBUF)],
            pltpu.VMEM((NODES_PER_W, IN_DIM), jnp.float32),
            [pltpu.SemaphoreType.DMA for _ in range(NBUF)],
        ],
    )
    def k(x_hbm, idx_hbm, out_hbm, idx_all, rows, outbuf, gsems):
        wid = lax.axis_index("c") * NUM_SUBCORES + lax.axis_index("s")

        pltpu.sync_copy(idx_hbm.at[pl.ds(wid * NODES_PER_W * DEG,
                                         NODES_PER_W * DEG)], idx_all)

        def fire(c, k_slot):
            idxv = idx_all.at[pl.ds(c * IDX_PER_CHUNK, IDX_PER_CHUNK)]
            pltpu.async_copy(x_hbm.at[idxv], rows[k_slot], gsems[k_slot])

        def wait_gather(c, k_slot):
            idxv = idx_all.at[pl.ds(c * IDX_PER_CHUNK, IDX_PER_CHUNK)]
            pltpu.make_async_copy(x_hbm.at[idxv], rows[k_slot],
                                  gsems[k_slot]).wait()

        def reduce_chunk(c, rowsv):
            @pl.loop(0, CHUNK)
            def _(n):
                base = n * DEG
                for g in range(GROUPS):
                    sl = pl.ds(g * LANES, LANES)
                    vals = [rowsv[base + j, sl] for j in range(DEG)]
                    outbuf[c * CHUNK + n, sl] = _tree_sum(vals)

        for k_slot in range(NBUF):
            fire(k_slot, k_slot)

        @pl.loop(0, NCH, step=NBUF)
        def _(c):
            for k_slot in range(NBUF):
                wait_gather(c + k_slot, k_slot)
                reduce_chunk(c + k_slot, rows[k_slot])

                @pl.when(c + k_slot + NBUF < NCH)
                def _():
                    fire(c + k_slot + NBUF, k_slot)

        pltpu.sync_copy(outbuf, out_hbm.at[pl.ds(wid * NODES_PER_W,
                                                 NODES_PER_W)])

    return k(x, idx_flat)


def _tc_linear(x, aggsum, W, b2):
    BLK = 2000

    def body(x_ref, a_ref, w_ref, b_ref, o_ref):
        w = w_ref[...]
        h1 = lax.dot_general(x_ref[...], w[:, :IN_DIM],
                             (((1,), (1,)), ((), ())),
                             precision=lax.Precision.HIGHEST,
                             preferred_element_type=jnp.float32)
        h2 = lax.dot_general(a_ref[...], w[:, IN_DIM:],
                             (((1,), (1,)), ((), ())),
                             precision=lax.Precision.HIGHEST,
                             preferred_element_type=jnp.float32)
        o_ref[...] = jnp.maximum(h1 + h2 * (1.0 / DEG) + b_ref[...], 0.0)

    return pl.pallas_call(
        body,
        grid=(N // BLK,),
        in_specs=[
            pl.BlockSpec((BLK, IN_DIM), lambda i: (i, 0)),
            pl.BlockSpec((BLK, IN_DIM), lambda i: (i, 0)),
            pl.BlockSpec((HIDDEN, 2 * IN_DIM), lambda i: (0, 0)),
            pl.BlockSpec((1, HIDDEN), lambda i: (0, 0)),
        ],
        out_specs=pl.BlockSpec((BLK, HIDDEN), lambda i: (i, 0)),
        out_shape=jax.ShapeDtypeStruct((N, HIDDEN), jnp.float32),
    )(x, aggsum, W, b2)


def kernel(x, adj, W, b):
    idx = adj.astype(jnp.int32).reshape(-1)
    pad = jnp.arange((NPAD - N) * DEG, dtype=jnp.int32) % N
    idx = jnp.concatenate([idx, pad])
    aggsum = _sc_neighbor_sum(x, idx)
    return _tc_linear(x, aggsum, W, b.reshape(1, HIDDEN))


# speedup vs baseline: 6.7457x; 1.0204x over previous
import functools

import jax
import jax.numpy as jnp
from jax import lax
from jax.experimental import pallas as pl
from jax.experimental.pallas import tpu as pltpu
from jax.experimental.pallas import tpu_sc as plsc

N = 10000
DEG = 32
IN_DIM = 128
HIDDEN = 128

NUM_CORES = 2
NUM_SUBCORES = 16
NW = NUM_CORES * NUM_SUBCORES
NODES_PER_W = 320
NPAD = NW * NODES_PER_W
CHUNK = 4
IDX_PER_CHUNK = CHUNK * DEG
NCH = NODES_PER_W // CHUNK
LAST_NODES = N - (NW - 1) * NODES_PER_W
LAST_NCH = LAST_NODES // CHUNK
LANES = 16
GROUPS = IN_DIM // LANES


def _tree_sum(vals):
    while len(vals) > 1:
        nxt = [vals[i] + vals[i + 1] for i in range(0, len(vals) - 1, 2)]
        if len(vals) % 2:
            nxt.append(vals[-1])
        vals = nxt
    return vals[0]


def _sc_neighbor_sum(x, idx_flat):
    mesh = plsc.VectorSubcoreMesh(core_axis_name="c", subcore_axis_name="s")
    NBUF = 4

    @functools.partial(
        pl.kernel,
        out_type=jax.ShapeDtypeStruct((NPAD, IN_DIM), jnp.float32),
        mesh=mesh,
        scratch_types=[
            pltpu.VMEM((NODES_PER_W * DEG,), jnp.int32),
            [pltpu.VMEM((IDX_PER_CHUNK, IN_DIM), jnp.float32)
             for _ in range(NBUF)],
            pltpu.VMEM((NODES_PER_W, IN_DIM), jnp.float32),
            [pltpu.SemaphoreType.DMA for _ in range(NBUF)],
        ],
    )
    def k(x_hbm, idx_hbm, out_hbm, idx_all, rows, outbuf, gsems):
        wid = lax.axis_index("c") * NUM_SUBCORES + lax.axis_index("s")
        is_last = wid == NW - 1
        nch = jnp.where(is_last, LAST_NCH, NCH)

        @pl.when(is_last)
        def _():
            pltpu.sync_copy(idx_hbm.at[pl.ds(wid * NODES_PER_W * DEG,
                                             LAST_NODES * DEG)],
                            idx_all.at[pl.ds(0, LAST_NODES * DEG)])

        @pl.when(jnp.logical_not(is_last))
        def _():
            pltpu.sync_copy(idx_hbm.at[pl.ds(wid * NODES_PER_W * DEG,
                                             NODES_PER_W * DEG)], idx_all)

        def fire(c, k_slot):
            idxv = idx_all.at[pl.ds(c * IDX_PER_CHUNK, IDX_PER_CHUNK)]
            pltpu.async_copy(x_hbm.at[idxv], rows[k_slot], gsems[k_slot])

        def wait_gather(c, k_slot):
            idxv = idx_all.at[pl.ds(c * IDX_PER_CHUNK, IDX_PER_CHUNK)]
            pltpu.make_async_copy(x_hbm.at[idxv], rows[k_slot],
                                  gsems[k_slot]).wait()

        def reduce_chunk(c, rowsv):
            @pl.loop(0, CHUNK)
            def _(n):
                base = n * DEG
                for g in range(GROUPS):
                    sl = pl.ds(g * LANES, LANES)
                    vals = [rowsv[base + j, sl] for j in range(DEG)]
                    outbuf[c * CHUNK + n, sl] = _tree_sum(vals)

        for k_slot in range(NBUF):
            fire(k_slot, k_slot)

        @pl.loop(0, nch, step=NBUF)
        def _(c):
            for k_slot in range(NBUF):
                wait_gather(c + k_slot, k_slot)
                reduce_chunk(c + k_slot, rows[k_slot])

                @pl.when(c + k_slot + NBUF < nch)
                def _():
                    fire(c + k_slot + NBUF, k_slot)

        @pl.when(is_last)
        def _():
            pltpu.sync_copy(outbuf.at[pl.ds(0, LAST_NODES)],
                            out_hbm.at[pl.ds(wid * NODES_PER_W, LAST_NODES)])

        @pl.when(jnp.logical_not(is_last))
        def _():
            pltpu.sync_copy(outbuf, out_hbm.at[pl.ds(wid * NODES_PER_W,
                                                     NODES_PER_W)])

    return k(x, idx_flat)


def _tc_linear(x, aggsum, W, b2):
    BLK = 2000

    def body(x_ref, a_ref, w_ref, b_ref, o_ref):
        w = w_ref[...]
        h1 = lax.dot_general(x_ref[...], w[:, :IN_DIM],
                             (((1,), (1,)), ((), ())),
                             precision=lax.Precision.HIGHEST,
                             preferred_element_type=jnp.float32)
        h2 = lax.dot_general(a_ref[...], w[:, IN_DIM:],
                             (((1,), (1,)), ((), ())),
                             precision=lax.Precision.HIGHEST,
                             preferred_element_type=jnp.float32)
        o_ref[...] = jnp.maximum(h1 + h2 * (1.0 / DEG) + b_ref[...], 0.0)

    return pl.pallas_call(
        body,
        grid=(N // BLK,),
        in_specs=[
            pl.BlockSpec((BLK, IN_DIM), lambda i: (i, 0)),
            pl.BlockSpec((BLK, IN_DIM), lambda i: (i, 0)),
            pl.BlockSpec((HIDDEN, 2 * IN_DIM), lambda i: (0, 0)),
            pl.BlockSpec((1, HIDDEN), lambda i: (0, 0)),
        ],
        out_specs=pl.BlockSpec((BLK, HIDDEN), lambda i: (i, 0)),
        out_shape=jax.ShapeDtypeStruct((N, HIDDEN), jnp.float32),
    )(x, aggsum, W, b2)


def kernel(x, adj, W, b):
    idx = adj.astype(jnp.int32).reshape(-1)
    aggsum = _sc_neighbor_sum(x, idx)
    return _tc_linear(x, aggsum, W, b.reshape(1, HIDDEN))
